# baseline (speedup 1.0000x reference)
import functools

import jax
import jax.numpy as jnp
from jax import lax
from jax.experimental import pallas as pl
from jax.experimental.pallas import tpu as pltpu
from jax.experimental.pallas import tpu_sc as plsc

_VOCAB = 100000
_DIM = 768
_B = 4 * 2048
_NC = 2
_NS = 16
_NW = _NC * _NS
_BPW = _B // _NW
_CHUNK = 32
_NCHUNK = _BPW // _CHUNK
_NBUF = 5
_AHEAD = 3


@jax.jit
def _sc_gather3(W0, W1, W2, idx_flat):
    mesh = plsc.VectorSubcoreMesh(
        core_axis_name="c", subcore_axis_name="s", num_cores=_NC,
        num_subcores=_NS)
    out_type = [jax.ShapeDtypeStruct((_B, _DIM), jnp.float32)] * 6

    @functools.partial(
        pl.kernel,
        mesh=mesh,
        out_type=out_type,
        scratch_types=(
            [pltpu.VMEM((_BPW,), jnp.int32)]
            + [pltpu.VMEM((_CHUNK, _DIM), jnp.float32)] * _NBUF
            + [pltpu.SemaphoreType.DMA] * (3 * _NBUF)
        ),
    )
    def body(w0, w1, w2, idx_hbm, o0, o1, o2, o3, o4, o5, idx_v, *rest):
        bufs = rest[:_NBUF]
        gsems = rest[_NBUF:2 * _NBUF]
        wsems_a = rest[2 * _NBUF:3 * _NBUF]
        wsems_b = rest[3 * _NBUF:]
        wid = lax.axis_index("s") * _NC + lax.axis_index("c")
        base = wid * _BPW
        pltpu.sync_copy(idx_hbm.at[pl.ds(base, _BPW)], idx_v)

        tables = (w0, w1, w2)
        outs_a = (o0, o1, o2)
        outs_b = (o5, o4, o3)
        tasks = [(t, c) for t in range(3) for c in range(_NCHUNK)]
        n = len(tasks)

        def start_gather(i):
            t, c = tasks[i]
            b = i % _NBUF
            return pltpu.async_copy(
                tables[t].at[idx_v.at[pl.ds(c * _CHUNK, _CHUNK)]],
                bufs[b], gsems[b])

        pend_g = [None] * _NBUF
        pend_w = [None] * _NBUF
        for j in range(min(_AHEAD, n)):
            pend_g[j % _NBUF] = start_gather(j)
        for i, (t, c) in enumerate(tasks):
            b = i % _NBUF
            sl = pl.ds(base + c * _CHUNK, _CHUNK)
            pend_g[b].wait()
            wa = pltpu.async_copy(bufs[b], outs_a[t].at[sl], wsems_a[b])
            wb = pltpu.async_copy(bufs[b], outs_b[t].at[sl], wsems_b[b])
            pend_w[b] = (wa, wb)
            k = i + _AHEAD
            if k < n:
                bk = k % _NBUF
                if pend_w[bk] is not None:
                    pend_w[bk][0].wait()
                    pend_w[bk][1].wait()
                pend_g[bk] = start_gather(k)
        for b in range(_NBUF):
            if pend_w[b] is not None:
                pend_w[b][0].wait()
                pend_w[b][1].wait()

    return body(W0, W1, W2, idx_flat)


def kernel(W0, W1, W2, inputs):
    idx_flat = inputs.reshape(-1).astype(jnp.int32)
    outs = _sc_gather3(W0, W1, W2, idx_flat)
    shape = inputs.shape + (_DIM,)
    return tuple(o.reshape(shape) for o in outs)


# speedup vs baseline: 1.0005x; 1.0005x over previous
import functools

import jax
import jax.numpy as jnp
from jax import lax
from jax.experimental import pallas as pl
from jax.experimental.pallas import tpu as pltpu
from jax.experimental.pallas import tpu_sc as plsc

_VOCAB = 100000
_DIM = 768
_B = 4 * 2048
_NC = 2
_NS = 16
_NW = _NC * _NS
_BPW = _B // _NW
_CHUNK = 32
_NCHUNK = _BPW // _CHUNK
_NBUF = 5
_AHEAD = 3


@jax.jit
def _sc_gather3(W0, W1, W2, idx2d):
    mesh = plsc.VectorSubcoreMesh(
        core_axis_name="c", subcore_axis_name="s", num_cores=_NC,
        num_subcores=_NS)
    out_type = [jax.ShapeDtypeStruct((_B, _DIM), jnp.float32)] * 6

    @functools.partial(
        pl.kernel,
        mesh=mesh,
        out_type=out_type,
        scratch_types=(
            [pltpu.VMEM((_BPW,), jnp.int32)]
            + [pltpu.VMEM((_CHUNK, _DIM), jnp.float32)] * _NBUF
            + [pltpu.SemaphoreType.DMA] * (3 * _NBUF)
        ),
    )
    def body(w0, w1, w2, idx_hbm, o0, o1, o2, o3, o4, o5, idx_v, *rest):
        bufs = rest[:_NBUF]
        gsems = rest[_NBUF:2 * _NBUF]
        wsems_a = rest[2 * _NBUF:3 * _NBUF]
        wsems_b = rest[3 * _NBUF:]
        wid = lax.axis_index("s") * _NC + lax.axis_index("c")
        base = wid * _BPW
        row = base // 2048
        col = base % 2048
        pltpu.sync_copy(idx_hbm.at[row, pl.ds(col, _BPW)], idx_v)

        tables = (w0, w1, w2)
        outs_a = (o0, o1, o2)
        outs_b = (o5, o4, o3)
        tasks = [(t, c) for t in range(3) for c in range(_NCHUNK)]
        n = len(tasks)

        def start_gather(i):
            t, c = tasks[i]
            b = i % _NBUF
            return pltpu.async_copy(
                tables[t].at[idx_v.at[pl.ds(c * _CHUNK, _CHUNK)]],
                bufs[b], gsems[b])

        pend_g = [None] * _NBUF
        pend_w = [None] * _NBUF
        for j in range(min(_AHEAD, n)):
            pend_g[j % _NBUF] = start_gather(j)
        for i, (t, c) in enumerate(tasks):
            b = i % _NBUF
            sl = pl.ds(base + c * _CHUNK, _CHUNK)
            pend_g[b].wait()
            wa = pltpu.async_copy(bufs[b], outs_a[t].at[sl], wsems_a[b])
            wb = pltpu.async_copy(bufs[b], outs_b[t].at[sl], wsems_b[b])
            pend_w[b] = (wa, wb)
            k = i + _AHEAD
            if k < n:
                bk = k % _NBUF
                if pend_w[bk] is not None:
                    pend_w[bk][0].wait()
                    pend_w[bk][1].wait()
                pend_g[bk] = start_gather(k)
        for b in range(_NBUF):
            if pend_w[b] is not None:
                pend_w[b][0].wait()
                pend_w[b][1].wait()

    return body(W0, W1, W2, idx2d)


def kernel(W0, W1, W2, inputs):
    outs = _sc_gather3(W0, W1, W2, inputs.astype(jnp.int32))
    shape = inputs.shape + (_DIM,)
    return tuple(o.reshape(shape) for o in outs)


# speedup vs baseline: 1.0010x; 1.0005x over previous
import functools

import jax
import jax.numpy as jnp
from jax import lax
from jax.experimental import pallas as pl
from jax.experimental.pallas import tpu as pltpu
from jax.experimental.pallas import tpu_sc as plsc

_VOCAB = 100000
_DIM = 768
_B = 4 * 2048
_NC = 2
_NS = 16
_NW = _NC * _NS
_BPW = _B // _NW
_CHUNK = 32
_NCHUNK = _BPW // _CHUNK
_NBUF = 5
_AHEAD = 3


@jax.jit
def _sc_gather3(W0, W1, W2, idx_flat):
    mesh = plsc.VectorSubcoreMesh(
        core_axis_name="c", subcore_axis_name="s", num_cores=_NC,
        num_subcores=_NS)
    out_type = [jax.ShapeDtypeStruct((_B, _DIM), jnp.float32)] * 6

    @functools.partial(
        pl.kernel,
        mesh=mesh,
        out_type=out_type,
        scratch_types=(
            [pltpu.VMEM((_BPW,), jnp.int32)]
            + [pltpu.VMEM((_CHUNK, _DIM), jnp.float32)] * _NBUF
            + [pltpu.SemaphoreType.DMA] * (3 * _NBUF)
        ),
    )
    def body(w0, w1, w2, idx_hbm, o0, o1, o2, o3, o4, o5, idx_v, *rest):
        bufs = rest[:_NBUF]
        gsems = rest[_NBUF:2 * _NBUF]
        wsems_a = rest[2 * _NBUF:3 * _NBUF]
        wsems_b = rest[3 * _NBUF:]
        wid = lax.axis_index("s") * _NC + lax.axis_index("c")
        base = wid * _BPW
        pltpu.sync_copy(idx_hbm.at[pl.ds(base, _BPW)], idx_v)

        tables = (w0, w1, w2)
        outs_a = (o0, o1, o2)
        outs_b = (o5, o4, o3)
        tasks = [(t, c) for t in range(3) for c in range(_NCHUNK)]
        n = len(tasks)

        def start_gather(i):
            t, c = tasks[i]
            b = i % _NBUF
            return pltpu.async_copy(
                tables[t].at[idx_v.at[pl.ds(c * _CHUNK, _CHUNK)]],
                bufs[b], gsems[b])

        pend_g = [None] * _NBUF
        pend_w = [None] * _NBUF
        for j in range(min(_AHEAD, n)):
            pend_g[j % _NBUF] = start_gather(j)
        for i, (t, c) in enumerate(tasks):
            b = i % _NBUF
            sl = pl.ds(base + c * _CHUNK, _CHUNK)
            pend_g[b].wait()
            wa = pltpu.async_copy(bufs[b], outs_a[t].at[sl], wsems_a[b])
            wb = pltpu.async_copy(bufs[b], outs_b[t].at[sl], wsems_b[b])
            pend_w[b] = (wa, wb)
            k = i + _AHEAD
            if k < n:
                bk = k % _NBUF
                if pend_w[bk] is not None:
                    pend_w[bk][0].wait()
                    pend_w[bk][1].wait()
                pend_g[bk] = start_gather(k)
        for b in range(_NBUF):
            if pend_w[b] is not None:
                pend_w[b][0].wait()
                pend_w[b][1].wait()

    return body(W0, W1, W2, idx_flat)


def kernel(W0, W1, W2, inputs):
    idx_flat = inputs.reshape(-1).astype(jnp.int32)
    outs = _sc_gather3(W0, W1, W2, idx_flat)
    shape = inputs.shape + (_DIM,)
    return tuple(o.reshape(shape) for o in outs)
